# Initial kernel scaffold; baseline (speedup 1.0000x reference)
import jax
import jax.numpy as jnp
from jax.experimental import pallas as pl


def kernel(x, emb, W, b):
    raise NotImplementedError("write your pallas kernel here")


# speedup vs baseline: 8.4884x; 8.4884x over previous
import functools

import jax
import jax.numpy as jnp
from jax import lax
from jax.experimental import pallas as pl
from jax.experimental.pallas import tpu as pltpu
from jax.experimental.pallas import tpu_sc as plsc

VOCAB = 1000000
EMB_DIM = 16
BATCH = 16384
HIST = 200

NC, NS, L = 2, 16, 16
NW = NC * NS
ROWS_PER_W = BATCH // NW
CH = 128
NCHUNK = ROWS_PER_W // CH
CHW = CH * HIST

V8 = VOCAB // 8
SCORE_BLK = 1000


def _score_body(e_ref, g_ref, o_ref):
    o_ref[...] = jnp.dot(e_ref[...], g_ref[...],
                         preferred_element_type=jnp.float32)


def _compute_scores(emb, W):
    emb2 = emb.reshape(V8, 128)
    G = jnp.kron(jnp.eye(8, dtype=jnp.float32), W.reshape(EMB_DIM, 1))
    scores8 = pl.pallas_call(
        _score_body,
        grid=(V8 // SCORE_BLK,),
        in_specs=[
            pl.BlockSpec((SCORE_BLK, 128), lambda i: (i, 0)),
            pl.BlockSpec((128, 8), lambda i: (0, 0)),
        ],
        out_specs=pl.BlockSpec((SCORE_BLK, 8), lambda i: (i, 0)),
        out_shape=jax.ShapeDtypeStruct((V8, 8), jnp.float32),
    )(emb2, G)
    return scores8.reshape(VOCAB)


def _sc_body(xf, scores_hbm, b16, out_hbm, scores_sh, idx_v, vals_v,
             b_v, outbuf, sem):
    cid = lax.axis_index("c")
    sid = lax.axis_index("s")
    wid = sid * NC + cid

    q = 62504
    last = VOCAB - 15 * q

    def _stage(total):
        off = 0
        for sz in (CHW, CHW, total - 2 * CHW):
            base = sid * q + off
            pltpu.sync_copy(scores_hbm.at[pl.ds(base, sz)],
                            vals_v.at[pl.ds(0, sz)])
            pltpu.sync_copy(vals_v.at[pl.ds(0, sz)],
                            scores_sh.at[pl.ds(base, sz)])
            off += sz

    @pl.when(sid < 15)
    def _():
        _stage(q)

    @pl.when(sid == 15)
    def _():
        _stage(last)

    pltpu.sync_copy(b16, b_v)
    plsc.subcore_barrier()

    bvec = b_v[...]
    lanes = lax.iota(jnp.int32, L)
    inv_hist = jnp.float32(1.0 / HIST)

    for ci in range(NCHUNK):
        chunk = wid * NCHUNK + ci
        row_base = chunk * CH
        pltpu.sync_copy(xf.at[pl.ds(chunk * CHW, CHW)], idx_v)
        pltpu.async_copy(scores_sh.at[idx_v], vals_v, sem).wait()

        for g in range(CH // L):
            base_vec = (g * L + lanes) * HIST

            def red(l, acc, base_vec=base_vec):
                v = plsc.load_gather(vals_v, [base_vec + l])
                return acc + v

            s = lax.fori_loop(0, HIST, red, jnp.zeros((L,), jnp.float32))
            z = s * inv_hist + bvec
            y = 1.0 / (1.0 + jnp.exp(-z))
            r = y * 10000.0
            t = r.astype(jnp.int32)
            tf = t.astype(jnp.float32)
            frac = r - tf
            odd = (t & 1) == 1
            up = (frac > 0.5) | ((frac == 0.5) & odd)
            outbuf[pl.ds(g * L, L)] = jnp.where(up, tf + 1.0, tf) / 10000.0

        pltpu.sync_copy(outbuf, out_hbm.at[pl.ds(row_base, CH)])


_sc_kernel = functools.partial(
    pl.kernel,
    out_type=jax.ShapeDtypeStruct((BATCH,), jnp.float32),
    mesh=plsc.VectorSubcoreMesh(core_axis_name="c", subcore_axis_name="s",
                                num_cores=NC, num_subcores=NS),
    scratch_types=[
        pltpu.VMEM_SHARED((VOCAB,), jnp.float32),
        pltpu.VMEM((CHW,), jnp.int32),
        pltpu.VMEM((CHW,), jnp.float32),
        pltpu.VMEM((L,), jnp.float32),
        pltpu.VMEM((CH,), jnp.float32),
        pltpu.SemaphoreType.DMA,
    ],
    compiler_params=pltpu.CompilerParams(needs_layout_passes=False),
)(_sc_body)


def kernel(x, emb, W, b):
    scores = _compute_scores(emb, W)
    xf = x.astype(jnp.int32).reshape(BATCH * HIST)
    b16 = jnp.broadcast_to(b.reshape(1).astype(jnp.float32), (L,))
    out = _sc_kernel(xf, scores, b16)
    return out.reshape(BATCH, 1)
